# baseline (speedup 1.0000x reference)
import functools

import jax
import jax.numpy as jnp
from jax import lax
from jax.experimental import pallas as pl
from jax.experimental.pallas import tpu as pltpu
from jax.experimental.pallas import tpu_sc as plsc

NC = 2
NS = 16
NW = NC * NS

GRP = 128
GRPS_PER_CHUNK = 8
CHUNK = GRP * GRPS_PER_CHUNK
LANES = 16
HID = 32


@functools.partial(jax.jit, static_argnames=("n_tasks", "n_chunks"))
def _gather_call(ids5, table, n_tasks, n_chunks):
    n_t = ids5.shape[0] * ids5.shape[2]
    n_tc = ids5.shape[1]
    tc_per_chunk = GRPS_PER_CHUNK
    blocks_per_t = n_tc // tc_per_chunk
    mesh = plsc.VectorSubcoreMesh(core_axis_name="c", subcore_axis_name="s")

    @functools.partial(
        pl.kernel,
        mesh=mesh,
        out_type=jax.ShapeDtypeStruct((n_t, 4, n_tc, 8, GRP), jnp.float32),
        scratch_types=[
            pltpu.VMEM((2, GRPS_PER_CHUNK, GRP), jnp.int32),
            pltpu.VMEM((2, CHUNK, HID), jnp.float32),
            pltpu.VMEM((2, 4, 4, 8, GRP), jnp.float32),
            pltpu.SemaphoreType.DMA,
            pltpu.SemaphoreType.DMA,
            pltpu.SemaphoreType.DMA,
            pltpu.SemaphoreType.DMA,
        ],
        compiler_params=pltpu.CompilerParams(
            use_tc_tiling_on_sc=False, needs_layout_passes=False,
            disable_bounds_checks=True),
    )
    def k(ids_hbm, table_hbm, out_hbm, idx_v, rows_v, rt_v,
          idx_sem, g_sem, st_sem0, st_sem1):
        wid = lax.axis_index("s") * NC + lax.axis_index("c")
        k0 = wid * n_chunks

        def task(g):
            kk = jnp.minimum(k0 + g, n_tasks - 1)
            t = kk // blocks_per_t
            bblk = kk % blocks_per_t
            return t, bblk

        def idx_copy(g, buf):
            t, bblk = task(g)
            return pltpu.make_async_copy(
                ids_hbm.at[t // 8, pl.ds(bblk * tc_per_chunk, tc_per_chunk),
                           t % 8, :],
                idx_v.at[buf], idx_sem)

        def gather_copies(buf):
            return [
                pltpu.make_async_copy(
                    table_hbm.at[idx_v.at[buf, j]],
                    rows_v.at[buf, pl.ds(j * GRP, GRP)],
                    g_sem)
                for j in range(GRPS_PER_CHUNK)
            ]

        def store_copy(g, hf, sem):
            t, bblk = task(g)
            return pltpu.make_async_copy(
                rt_v.at[hf],
                out_hbm.at[t, :, pl.ds(bblk * tc_per_chunk + hf * 4, 4)],
                sem)

        iota = lax.iota(jnp.int32, LANES)
        hvecs = [jnp.full((LANES,), h, jnp.int32) for h in range(HID)]

        def transpose_half(buf, hf):
            src = rows_v.at[buf]

            def body(b8, _):
                rbase = hf * 512 + b8 * GRP
                for c16 in range(8):
                    idx0 = rbase + c16 * LANES + iota
                    for h in range(HID):
                        vals = plsc.load_gather(src, [idx0, hvecs[h]])
                        rt_v[hf, h // 8, b8, h % 8,
                             pl.ds(c16 * LANES, LANES)] = vals
                return 0

            lax.fori_loop(0, 4, body, 0, unroll=False)

        def do_chunk(g, buf):
            @pl.when(g < n_chunks - 1)
            def _():
                idx_copy(g + 1, 1 - buf).wait()
            for c in gather_copies(buf):
                c.wait()

            @pl.when(g < n_chunks - 1)
            def _():
                for c in gather_copies(1 - buf):
                    c.start()

            @pl.when(g < n_chunks - 2)
            def _():
                idx_copy(g + 2, buf).start()

            @pl.when(g >= 1)
            def _():
                store_copy(g - 1, 0, st_sem0).wait()
            transpose_half(buf, 0)
            store_copy(g, 0, st_sem0).start()

            @pl.when(g >= 1)
            def _():
                store_copy(g - 1, 1, st_sem1).wait()
            transpose_half(buf, 1)
            store_copy(g, 1, st_sem1).start()

        idx_copy(0, 0).start()
        idx_copy(1, 1).start()
        idx_copy(0, 0).wait()
        for c in gather_copies(0):
            c.start()

        def pair(p, _):
            do_chunk(2 * p, 0)
            do_chunk(2 * p + 1, 1)
            return 0

        lax.fori_loop(0, n_chunks // 2, pair, 0, unroll=False)
        store_copy(n_chunks - 1, 0, st_sem0).wait()
        store_copy(n_chunks - 1, 1, st_sem1).wait()

    return k(ids5, table)


def kernel(input_ids, table):
    b, t = input_ids.shape
    hidden = table.shape[1]
    assert (b, t, hidden) == (16384, 200, 32)
    n_tasks = t * (b // CHUNK)
    assert n_tasks % NW == 0
    n_chunks = n_tasks // NW
    ids5 = input_ids.reshape(b // GRP, GRP, t // 8, 8).transpose(2, 0, 3, 1)
    out5 = _gather_call(ids5, table, n_tasks, n_chunks)
    out = out5.transpose(0, 1, 3, 2, 4).reshape(t, hidden, b).transpose(2, 0, 1)
    return out


# speedup vs baseline: 2.2344x; 2.2344x over previous
import functools

import jax
import jax.numpy as jnp
import numpy as np
from jax import lax
from jax.experimental import pallas as pl
from jax.experimental.pallas import tpu as pltpu
from jax.experimental.pallas import tpu_sc as plsc

NC = 2
NS = 16
NW = NC * NS

GRP = 128
GRPS_PER_CHUNK = 8
CHUNK = GRP * GRPS_PER_CHUNK
LANES = 16
HID = 32
RT = 4 * 4 * 8 * GRP


@functools.partial(jax.jit, static_argnames=("n_tasks", "n_chunks"))
def _gather_call(ids5, table, n_tasks, n_chunks):
    n_t = ids5.shape[0] * ids5.shape[2]
    n_tc = ids5.shape[1]
    tc_per_chunk = GRPS_PER_CHUNK
    blocks_per_t = n_tc // tc_per_chunk
    out_flat = n_t * 4 * n_tc * 8 * GRP
    mesh = plsc.VectorSubcoreMesh(core_axis_name="c", subcore_axis_name="s")

    @functools.partial(
        pl.kernel,
        mesh=mesh,
        out_type=jax.ShapeDtypeStruct((out_flat,), jnp.float32),
        scratch_types=[
            pltpu.VMEM((2, GRPS_PER_CHUNK, GRP), jnp.int32),
            pltpu.VMEM((2, CHUNK, HID), jnp.float32),
            pltpu.VMEM((2, RT), jnp.float32),
            pltpu.SemaphoreType.DMA,
            pltpu.SemaphoreType.DMA,
            pltpu.SemaphoreType.DMA,
            pltpu.SemaphoreType.DMA,
        ],
        compiler_params=pltpu.CompilerParams(
            use_tc_tiling_on_sc=False, needs_layout_passes=False,
            disable_bounds_checks=True),
    )
    def k(ids_hbm, table_hbm, out_hbm, idx_v, rows_v, rt_v,
          idx_sem, g_sem, st_sem0, st_sem1):
        wid = lax.axis_index("s") * NC + lax.axis_index("c")
        k0 = wid * n_chunks

        def task(g):
            kk = jnp.minimum(k0 + g, n_tasks - 1)
            t = kk // blocks_per_t
            bblk = kk % blocks_per_t
            return t, bblk

        def idx_copy(g, buf):
            t, bblk = task(g)
            return pltpu.make_async_copy(
                ids_hbm.at[t // 8, pl.ds(bblk * tc_per_chunk, tc_per_chunk),
                           t % 8, :],
                idx_v.at[buf], idx_sem)

        def gather_copies(buf):
            return [
                pltpu.make_async_copy(
                    table_hbm.at[idx_v.at[buf, j]],
                    rows_v.at[buf, pl.ds(j * GRP, GRP)],
                    g_sem)
                for j in range(GRPS_PER_CHUNK)
            ]

        def store_copies(g, hf, sem):
            t, bblk = task(g)
            cps = []
            for hblk in range(4):
                off = (((t * 4 + hblk) * n_tc)
                       + bblk * tc_per_chunk + hf * 4) * 1024
                cps.append(pltpu.make_async_copy(
                    rt_v.at[hf, pl.ds(hblk * 4096, 4096)],
                    out_hbm.at[pl.ds(off, 4096)], sem))
            return cps

        iota = lax.iota(jnp.int32, LANES)

        def transpose_half(buf, hf):
            src = rows_v.at[buf]
            dst = rt_v.at[hf]

            def body(hb, _):
                b8 = hb >> 5
                h = hb & (HID - 1)
                hrot = (iota + h) & (HID - 1)
                hflat = ((hrot >> 3) << 12) + ((hrot & 7) << 7) + iota
                r00 = (hf * 4 + b8) * GRP
                sidx = b8 * 1024 + hflat
                for c16 in range(8):
                    vals = plsc.load_gather(
                        src, [r00 + c16 * LANES + iota, hrot])
                    plsc.store_scatter(dst, [sidx + c16 * LANES], vals)
                return 0

            lax.fori_loop(0, 4 * HID, body, 0, unroll=False)

        def do_chunk(g, buf):
            @pl.when(g < n_chunks - 1)
            def _():
                idx_copy(g + 1, 1 - buf).wait()
            for c in gather_copies(buf):
                c.wait()

            @pl.when(g < n_chunks - 1)
            def _():
                for c in gather_copies(1 - buf):
                    c.start()

            @pl.when(g < n_chunks - 2)
            def _():
                idx_copy(g + 2, buf).start()

            @pl.when(g >= 1)
            def _():
                for c in store_copies(g - 1, 0, st_sem0):
                    c.wait()
            transpose_half(buf, 0)
            for c in store_copies(g, 0, st_sem0):
                c.start()

            @pl.when(g >= 1)
            def _():
                for c in store_copies(g - 1, 1, st_sem1):
                    c.wait()
            transpose_half(buf, 1)
            for c in store_copies(g, 1, st_sem1):
                c.start()

        idx_copy(0, 0).start()
        idx_copy(1, 1).start()
        idx_copy(0, 0).wait()
        for c in gather_copies(0):
            c.start()

        def pair(p, _):
            do_chunk(2 * p, 0)
            do_chunk(2 * p + 1, 1)
            return 0

        lax.fori_loop(0, n_chunks // 2, pair, 0, unroll=False)
        for c in store_copies(n_chunks - 1, 0, st_sem0):
            c.wait()
        for c in store_copies(n_chunks - 1, 1, st_sem1):
            c.wait()

    return k(ids5, table)


def kernel(input_ids, table):
    b, t = input_ids.shape
    hidden = table.shape[1]
    assert (b, t, hidden) == (16384, 200, 32)
    n_tasks = t * (b // CHUNK)
    assert n_tasks % NW == 0
    n_chunks = n_tasks // NW
    ids5 = input_ids.reshape(b // GRP, GRP, t // 8, 8).transpose(2, 0, 3, 1)
    outf = _gather_call(ids5, table, n_tasks, n_chunks)
    out5 = outf.reshape(t, 4, b // GRP, 8, GRP)
    out = out5.transpose(0, 1, 3, 2, 4).reshape(t, hidden, b).transpose(2, 0, 1)
    return out


# speedup vs baseline: 2.2427x; 1.0037x over previous
import functools

import jax
import jax.numpy as jnp
import numpy as np
from jax import lax
from jax.experimental import pallas as pl
from jax.experimental.pallas import tpu as pltpu
from jax.experimental.pallas import tpu_sc as plsc

NC = 2
NS = 16
NW = NC * NS

GRP = 128
GRPS_PER_CHUNK = 8
CHUNK = GRP * GRPS_PER_CHUNK
LANES = 16
HID = 32
RT = 4 * 4 * 8 * GRP


@functools.partial(jax.jit, static_argnames=("n_tasks", "n_chunks"))
def _gather_call(ids5, table, n_tasks, n_chunks):
    n_t = ids5.shape[0] * ids5.shape[2]
    n_tc = ids5.shape[1]
    tc_per_chunk = GRPS_PER_CHUNK
    blocks_per_t = n_tc // tc_per_chunk
    out_flat = n_t * 4 * n_tc * 8 * GRP
    mesh = plsc.VectorSubcoreMesh(core_axis_name="c", subcore_axis_name="s")

    @functools.partial(
        pl.kernel,
        mesh=mesh,
        out_type=jax.ShapeDtypeStruct((out_flat,), jnp.float32),
        scratch_types=[
            pltpu.VMEM((2, GRPS_PER_CHUNK, GRP), jnp.int32),
            pltpu.VMEM((2, CHUNK, HID), jnp.float32),
            pltpu.VMEM((2, RT), jnp.float32),
            pltpu.SemaphoreType.DMA,
            pltpu.SemaphoreType.DMA,
            pltpu.SemaphoreType.DMA,
            pltpu.SemaphoreType.DMA,
        ],
        compiler_params=pltpu.CompilerParams(
            use_tc_tiling_on_sc=False, needs_layout_passes=False,
            disable_bounds_checks=True),
    )
    def k(ids_hbm, table_hbm, out_hbm, idx_v, rows_v, rt_v,
          idx_sem, g_sem, st_sem0, st_sem1):
        wid = lax.axis_index("s") * NC + lax.axis_index("c")
        k0 = wid * n_chunks

        def task(g):
            kk = jnp.minimum(k0 + g, n_tasks - 1)
            t = kk // blocks_per_t
            bblk = kk % blocks_per_t
            return t, bblk

        def idx_copy(g, buf):
            t, bblk = task(g)
            return pltpu.make_async_copy(
                ids_hbm.at[t // 8, pl.ds(bblk * tc_per_chunk, tc_per_chunk),
                           t % 8, :],
                idx_v.at[buf], idx_sem)

        def gather_copies(buf):
            return [
                pltpu.make_async_copy(
                    table_hbm.at[idx_v.at[buf, j]],
                    rows_v.at[buf, pl.ds(j * GRP, GRP)],
                    g_sem)
                for j in range(GRPS_PER_CHUNK)
            ]

        def gather_drain(buf):
            return pltpu.make_async_copy(
                table_hbm.at[pl.ds(0, CHUNK)], rows_v.at[buf], g_sem)

        def store_drain(hf, sem):
            return pltpu.make_async_copy(
                out_hbm.at[pl.ds(0, RT)], rt_v.at[hf], sem)

        def store_copies(g, hf, sem):
            t, bblk = task(g)
            cps = []
            for hblk in range(4):
                off = (((t * 4 + hblk) * n_tc)
                       + bblk * tc_per_chunk + hf * 4) * 1024
                cps.append(pltpu.make_async_copy(
                    rt_v.at[hf, pl.ds(hblk * 4096, 4096)],
                    out_hbm.at[pl.ds(off, 4096)], sem))
            return cps

        iota = lax.iota(jnp.int32, LANES)

        def transpose_half(buf, hf):
            src = rows_v.at[buf]
            dst = rt_v.at[hf]

            def body(hb, _):
                b8 = hb >> 5
                h = hb & (HID - 1)
                hrot = (iota + h) & (HID - 1)
                hflat = ((hrot >> 3) << 12) + ((hrot & 7) << 7) + iota
                r00 = (hf * 4 + b8) * GRP
                sidx = b8 * 1024 + hflat
                for c16 in range(8):
                    vals = plsc.load_gather(
                        src, [r00 + c16 * LANES + iota, hrot])
                    plsc.store_scatter(dst, [sidx + c16 * LANES], vals)
                return 0

            lax.fori_loop(0, 4 * HID, body, 0, unroll=False)

        def do_chunk(g, buf):
            @pl.when(g < n_chunks - 1)
            def _():
                idx_copy(g + 1, 1 - buf).wait()
            gather_drain(buf).wait()

            @pl.when(g < n_chunks - 1)
            def _():
                for c in gather_copies(1 - buf):
                    c.start()

            @pl.when(g < n_chunks - 2)
            def _():
                idx_copy(g + 2, buf).start()

            @pl.when(g >= 1)
            def _():
                store_drain(0, st_sem0).wait()
            transpose_half(buf, 0)
            for c in store_copies(g, 0, st_sem0):
                c.start()

            @pl.when(g >= 1)
            def _():
                store_drain(1, st_sem1).wait()
            transpose_half(buf, 1)
            for c in store_copies(g, 1, st_sem1):
                c.start()

        idx_copy(0, 0).start()
        idx_copy(1, 1).start()
        idx_copy(0, 0).wait()
        for c in gather_copies(0):
            c.start()

        def pair(p, _):
            do_chunk(2 * p, 0)
            do_chunk(2 * p + 1, 1)
            return 0

        lax.fori_loop(0, n_chunks // 2, pair, 0, unroll=False)
        store_drain(0, st_sem0).wait()
        store_drain(1, st_sem1).wait()

    return k(ids5, table)


def kernel(input_ids, table):
    b, t = input_ids.shape
    hidden = table.shape[1]
    assert (b, t, hidden) == (16384, 200, 32)
    n_tasks = t * (b // CHUNK)
    assert n_tasks % NW == 0
    n_chunks = n_tasks // NW
    ids5 = input_ids.reshape(b // GRP, GRP, t // 8, 8).transpose(2, 0, 3, 1)
    outf = _gather_call(ids5, table, n_tasks, n_chunks)
    out5 = outf.reshape(t, 4, b // GRP, 8, GRP)
    out = out5.transpose(0, 1, 3, 2, 4).reshape(t, hidden, b).transpose(2, 0, 1)
    return out


# speedup vs baseline: 3.7161x; 1.6570x over previous
import functools

import jax
import jax.numpy as jnp
import numpy as np
from jax import lax
from jax.experimental import pallas as pl
from jax.experimental.pallas import tpu as pltpu
from jax.experimental.pallas import tpu_sc as plsc

NC = 2
NS = 16
NW = NC * NS

GRP = 128
GRPS_PER_CHUNK = 8
CHUNK = GRP * GRPS_PER_CHUNK
LANES = 16
HID = 32
RT = 4 * 4 * 8 * GRP


@functools.partial(jax.jit, static_argnames=("n_tasks", "n_chunks"))
def _gather_call(ids5, table, n_tasks, n_chunks):
    n_t = ids5.shape[0] * ids5.shape[2]
    n_tc = ids5.shape[1]
    tc_per_chunk = GRPS_PER_CHUNK
    blocks_per_t = n_tc // tc_per_chunk
    out_flat = n_t * 4 * n_tc * 8 * GRP
    mesh = plsc.VectorSubcoreMesh(core_axis_name="c", subcore_axis_name="s")

    @functools.partial(
        pl.kernel,
        mesh=mesh,
        out_type=jax.ShapeDtypeStruct((out_flat,), jnp.float32),
        scratch_types=[
            pltpu.VMEM((2, GRPS_PER_CHUNK, GRP), jnp.int32),
            pltpu.VMEM((2, CHUNK, HID), jnp.float32),
            pltpu.VMEM((2, RT), jnp.float32),
            pltpu.SemaphoreType.DMA,
            pltpu.SemaphoreType.DMA,
            pltpu.SemaphoreType.DMA,
            pltpu.SemaphoreType.DMA,
        ],
        compiler_params=pltpu.CompilerParams(
            use_tc_tiling_on_sc=False, needs_layout_passes=False,
            disable_bounds_checks=True),
    )
    def k(ids_hbm, table_hbm, out_hbm, idx_v, rows_v, rt_v,
          idx_sem, g_sem, st_sem0, st_sem1):
        wid = lax.axis_index("s") * NC + lax.axis_index("c")
        k0 = wid * n_chunks

        def task(g):
            kk = jnp.minimum(k0 + g, n_tasks - 1)
            t = kk // blocks_per_t
            bblk = kk % blocks_per_t
            return t, bblk

        def idx_copy(g, buf):
            t, bblk = task(g)
            return pltpu.make_async_copy(
                ids_hbm.at[t // 8, pl.ds(bblk * tc_per_chunk, tc_per_chunk),
                           t % 8, :],
                idx_v.at[buf], idx_sem)

        def gather_copies(buf):
            return [
                pltpu.make_async_copy(
                    table_hbm.at[idx_v.at[buf, j]],
                    rows_v.at[buf, pl.ds(j * GRP, GRP)],
                    g_sem)
                for j in range(GRPS_PER_CHUNK)
            ]

        def gather_drain(buf):
            return pltpu.make_async_copy(
                table_hbm.at[pl.ds(0, CHUNK)], rows_v.at[buf], g_sem)

        def store_drain(hf, sem):
            return pltpu.make_async_copy(
                out_hbm.at[pl.ds(0, RT)], rt_v.at[hf], sem)

        def store_copies(g, hf, sem):
            t, bblk = task(g)
            cps = []
            for hblk in range(4):
                off = (((t * 4 + hblk) * n_tc)
                       + bblk * tc_per_chunk + hf * 4) * 1024
                cps.append(pltpu.make_async_copy(
                    rt_v.at[hf, pl.ds(hblk * 4096, 4096)],
                    out_hbm.at[pl.ds(off, 4096)], sem))
            return cps

        iota = lax.iota(jnp.int32, LANES)

        def transpose_half(buf, hf):
            src = rows_v.at[buf]
            dst = rt_v.at[hf]

            @functools.partial(plsc.parallel_loop, 0, 4 * HID, unroll=4)
            def body(hb):
                b8 = hb >> 5
                h = hb & (HID - 1)
                hrot = (iota + h) & (HID - 1)
                hflat = ((hrot >> 3) << 12) + ((hrot & 7) << 7) + iota
                r00 = (hf * 4 + b8) * GRP
                sidx = b8 * 1024 + hflat
                for c16 in range(8):
                    vals = plsc.load_gather(
                        src, [r00 + c16 * LANES + iota, hrot])
                    plsc.store_scatter(dst, [sidx + c16 * LANES], vals)

        def do_chunk(g, buf):
            @pl.when(g < n_chunks - 1)
            def _():
                idx_copy(g + 1, 1 - buf).wait()
            gather_drain(buf).wait()

            @pl.when(g < n_chunks - 1)
            def _():
                for c in gather_copies(1 - buf):
                    c.start()

            @pl.when(g < n_chunks - 2)
            def _():
                idx_copy(g + 2, buf).start()

            @pl.when(g >= 1)
            def _():
                store_drain(0, st_sem0).wait()
            transpose_half(buf, 0)
            for c in store_copies(g, 0, st_sem0):
                c.start()

            @pl.when(g >= 1)
            def _():
                store_drain(1, st_sem1).wait()
            transpose_half(buf, 1)
            for c in store_copies(g, 1, st_sem1):
                c.start()

        idx_copy(0, 0).start()
        idx_copy(1, 1).start()
        idx_copy(0, 0).wait()
        for c in gather_copies(0):
            c.start()

        def pair(p, _):
            do_chunk(2 * p, 0)
            do_chunk(2 * p + 1, 1)
            return 0

        lax.fori_loop(0, n_chunks // 2, pair, 0, unroll=False)
        store_drain(0, st_sem0).wait()
        store_drain(1, st_sem1).wait()

    return k(ids5, table)


def kernel(input_ids, table):
    b, t = input_ids.shape
    hidden = table.shape[1]
    assert (b, t, hidden) == (16384, 200, 32)
    n_tasks = t * (b // CHUNK)
    assert n_tasks % NW == 0
    n_chunks = n_tasks // NW
    ids5 = input_ids.reshape(b // GRP, GRP, t // 8, 8).transpose(2, 0, 3, 1)
    outf = _gather_call(ids5, table, n_tasks, n_chunks)
    out5 = outf.reshape(t, 4, b // GRP, 8, GRP)
    out = out5.transpose(0, 1, 3, 2, 4).reshape(t, hidden, b).transpose(2, 0, 1)
    return out
